# Initial kernel scaffold; baseline (speedup 1.0000x reference)
import jax
import jax.numpy as jnp
from jax.experimental import pallas as pl


def kernel(proj, hm, cluster_center, cluster_ind):
    raise NotImplementedError("write your pallas kernel here")


# speedup vs baseline: 3.3386x; 3.3386x over previous
import jax
import jax.numpy as jnp
from jax.experimental import pallas as pl
from jax.experimental.pallas import tpu as pltpu

EPS = 1e-8

_H = 1024
_W = 1024
_C = 16
_BH = 64
_GRID = _H // _BH


def _body(center_ref, f_ref, hm_ref, out_ref):
    i = pl.program_id(0)

    c = center_ref[0, :]
    cn = c / (jnp.sqrt(jnp.sum(c * c)) + EPS)

    f = f_ref[...]
    ss = jnp.sum(f * f, axis=0)
    dot = jnp.sum(f * cn[:, None, None], axis=0)
    sim = dot / (jnp.sqrt(ss) + EPS)
    d = sim - hm_ref[...]
    part = jnp.sum(d * d)

    @pl.when(i == 0)
    def _init():
        out_ref[...] = jnp.zeros_like(out_ref)

    out_ref[...] += part.reshape(1, 1)

    @pl.when(i == _GRID - 1)
    def _final():
        out_ref[...] *= 1.0 / (_H * _W)


def kernel(proj, hm, cluster_center, cluster_ind):
    center = jnp.take(cluster_center, cluster_ind, axis=0)
    center = jax.lax.stop_gradient(center).reshape(1, _C)
    f = proj.reshape(_C, _H, _W)
    hm2 = hm.reshape(_H, _W)

    out = pl.pallas_call(
        _body,
        grid=(_GRID,),
        in_specs=[
            pl.BlockSpec((1, _C), lambda i: (0, 0)),
            pl.BlockSpec((_C, _BH, _W), lambda i: (0, i, 0)),
            pl.BlockSpec((_BH, _W), lambda i: (i, 0)),
        ],
        out_specs=pl.BlockSpec((1, 1), lambda i: (0, 0)),
        out_shape=jax.ShapeDtypeStruct((1, 1), jnp.float32),
    )(center, f, hm2)

    loss = out[0, 0]
    return (loss, loss * 0.0, loss)
